# baseline (speedup 1.0000x reference)
import functools

import jax
import jax.numpy as jnp
from jax import lax
from jax.experimental import pallas as pl
from jax.experimental.pallas import tpu as pltpu
from jax.experimental.pallas import tpu_sc as plsc

B, N, C = 2, 4096, 1024
H, DH = 16, 64
NH, NB = 4, 8
G = B * NH * NB
R = B * N

BLK = 512
BLK34 = 512
NEG = -3.4028235e38

NC, NS = 2, 16
NW = NC * NS
RPW = R // NW
KCH = 32
NHSC = 2
NCH = RPW // KCH


def _ln(x, g, b, eps=1e-5):
    mu = jnp.mean(x, axis=-1, keepdims=True)
    var = jnp.mean((x - mu) ** 2, axis=-1, keepdims=True)
    return (x - mu) * jax.lax.rsqrt(var + eps) * g + b


def _p1a_body(x_ref, g_ref, b_ref, rot_ref, xn_ref, gid_ref, p23_ref,
              ca_ref):
    i = pl.program_id(0)
    b = i // (N // BLK)
    xn = _ln(x_ref[...], g_ref[...], b_ref[...])
    xn_ref[...] = xn
    rvT = lax.dot_general(rot_ref[...], xn, (((0,), (1,)), ((), ())),
                          preferred_element_type=jnp.float32)
    iota8 = lax.broadcasted_iota(jnp.int32, (NB, BLK), 0)
    ohs = []
    for h in range(NH):
        sub = rvT[h * (NB // 2):(h + 1) * (NB // 2), :]
        full = jnp.concatenate([sub, -sub], axis=0)
        mx = jnp.max(full, axis=0, keepdims=True)
        bid = jnp.min(jnp.where(full >= mx, iota8, NB), axis=0)
        gid_ref[h, :] = b * (NH * NB) + h * NB + bid
        ohs.append((iota8 == bid[None, :]).astype(jnp.float32))
    for h in range(NH, 8):
        gid_ref[h, :] = jnp.zeros((BLK,), jnp.int32)
    ohT = jnp.concatenate(ohs, axis=0)
    cc = jnp.broadcast_to(jnp.sum(ohT, axis=1, keepdims=True),
                          (NH * NB, 128))
    pc = lax.dot_general(ohT[NHSC * NB:].astype(jnp.bfloat16),
                         xn.astype(jnp.bfloat16), (((1,), (0,)), ((), ())),
                         preferred_element_type=jnp.float32)

    @pl.when(i % (N // BLK) == 0)
    def _init():
        p23_ref[0] = pc
        ca_ref[0] = cc

    @pl.when(i % (N // BLK) != 0)
    def _acc():
        p23_ref[0] += pc
        ca_ref[0] += cc


def _phase1a(x2d, g1, b1, rotflat):
    return pl.pallas_call(
        _p1a_body,
        grid=(R // BLK,),
        in_specs=[
            pl.BlockSpec((BLK, C), lambda i: (i, 0)),
            pl.BlockSpec((1, C), lambda i: (0, 0)),
            pl.BlockSpec((1, C), lambda i: (0, 0)),
            pl.BlockSpec((C, 16), lambda i: (0, 0)),
        ],
        out_specs=[
            pl.BlockSpec((BLK, C), lambda i: (i, 0)),
            pl.BlockSpec((8, BLK), lambda i: (0, i)),
            pl.BlockSpec((1, (NH - NHSC) * NB, C),
                         lambda i: (i // (N // BLK), 0, 0)),
            pl.BlockSpec((1, NH * NB, 128), lambda i: (i // (N // BLK), 0, 0)),
        ],
        out_shape=[
            jax.ShapeDtypeStruct((R, C), jnp.float32),
            jax.ShapeDtypeStruct((8, R), jnp.int32),
            jax.ShapeDtypeStruct((B, (NH - NHSC) * NB, C), jnp.float32),
            jax.ShapeDtypeStruct((B, NH * NB, 128), jnp.float32),
        ],
    )(x2d, g1, b1, rotflat)


def _sc_pool_body(xn_hbm, gid_hbm, outp_hbm,
                  xbuf, gb0, gb1, acc_p, sems):
    c = lax.axis_index("c")
    s = lax.axis_index("s")
    w = s * NC + c
    gbufs = (gb0, gb1)
    boff = (w // (NW // B)) * (NH * NB)

    cp0 = pltpu.async_copy(gid_hbm.at[0, pl.ds(w * RPW, RPW)], gb0,
                           sems.at[2])
    cp1 = pltpu.async_copy(gid_hbm.at[1, pl.ds(w * RPW, RPW)], gb1,
                           sems.at[3])
    first = pltpu.async_copy(xn_hbm.at[pl.ds(w * RPW, KCH)],
                             xbuf.at[0], sems.at[0])

    z16 = jnp.zeros((16,), jnp.float32)

    def zrow(r, _):
        def zcol(t, _):
            acc_p[r, pl.ds(t * 16, 16)] = z16
            return ()
        lax.fori_loop(0, C // 16, zcol, (), unroll=8)
        return ()

    lax.fori_loop(0, NHSC * NB, zrow, ())

    iota16 = lax.iota(jnp.int32, 16)
    cp0.wait()
    cp1.wait()

    for i in range(NCH):
        if i + 1 < NCH:
            nxt = pltpu.async_copy(
                xn_hbm.at[pl.ds(w * RPW + (i + 1) * KCH, KCH)],
                xbuf.at[(i + 1) % 2], sems.at[(i + 1) % 2])
        if i == 0:
            first.wait()
        else:
            prev.wait()
        for grp in range(KCH // 16):
            gvs = [gbufs[h][pl.ds(i * KCH + grp * 16, 16)].astype(jnp.float32)
                   for h in range(NHSC)]

            def row_body(l, _):
                js = [(jnp.sum(jnp.where(iota16 == l, gvs[h], 0.0))
                       .astype(jnp.int32) - boff) for h in range(NHSC)]
                r = grp * 16 + l

                def col_body(t, _):
                    x16 = xbuf[i % 2, r, pl.ds(t * 16, 16)]
                    for h in range(NHSC):
                        plsc.addupdate(acc_p.at[js[h], pl.ds(t * 16, 16)],
                                       x16)
                    return ()

                lax.fori_loop(0, C // 16, col_body, (), unroll=4)
                return ()

            lax.fori_loop(0, 16, row_body, ())
        if i + 1 < NCH:
            prev = nxt

    pltpu.sync_copy(acc_p, outp_hbm.at[w])


def _sc_pool(xn2d, gid8):
    mesh = plsc.VectorSubcoreMesh(core_axis_name="c", subcore_axis_name="s",
                                  num_cores=NC, num_subcores=NS)
    f = pl.kernel(
        _sc_pool_body,
        out_type=jax.ShapeDtypeStruct((NW, NHSC * NB, C), jnp.float32),
        mesh=mesh,
        compiler_params=pltpu.CompilerParams(needs_layout_passes=False),
        scratch_types=[
            pltpu.VMEM((2, KCH, C), jnp.float32),
            pltpu.VMEM((RPW,), jnp.int32),
            pltpu.VMEM((RPW,), jnp.int32),
            pltpu.VMEM((NHSC * NB, C), jnp.float32),
            pltpu.SemaphoreType.DMA((4,)),
        ],
    )
    return f(xn2d, gid8)


def _p2_body(pp_ref, p23_ref, ca_ref, wkv_ref, w1f_ref, w2f_ref,
             k_ref, v_ref, cnt_ref, w1b_ref, w2b_ref):
    w1b_ref[...] = w1f_ref[...].astype(jnp.bfloat16)
    w2b_ref[...] = w2f_ref[...].astype(jnp.bfloat16)
    halves = []
    for b in range(B):
        lo = b * (NW // B)
        acc = pp_ref[lo]
        for i in range(lo + 1, lo + NW // B):
            acc = acc + pp_ref[i]
        halves.append(jnp.concatenate([acc, p23_ref[b]], axis=0))
    pooled = jnp.concatenate(halves, axis=0)
    cnt = jnp.concatenate([ca_ref[0, :, 0:16], ca_ref[1, :, 0:16]], axis=0)
    cnt_ref[...] = cnt
    rp = pooled * (1.0 / (cnt[:, 0:1] + 1e-20))
    kv = lax.dot_general(rp, wkv_ref[...], (((1,), (1,)), ((), ())),
                         preferred_element_type=jnp.float32)
    k_ref[...] = kv[:, :C]
    v_ref[...] = kv[:, C:]


def _phase2(pp, p23, ca, Wkv, fc1_w, fc2_w):
    return pl.pallas_call(
        _p2_body,
        out_shape=[
            jax.ShapeDtypeStruct((G, C), jnp.float32),
            jax.ShapeDtypeStruct((G, C), jnp.float32),
            jax.ShapeDtypeStruct((G, 16), jnp.float32),
            jax.ShapeDtypeStruct((4 * C, C), jnp.bfloat16),
            jax.ShapeDtypeStruct((C, 4 * C), jnp.bfloat16),
        ],
        compiler_params=pltpu.CompilerParams(
            vmem_limit_bytes=128 * 1024 * 1024),
    )(pp, p23, ca, Wkv, fc1_w, fc2_w)


def _p34_body(x_ref, xn_ref, wq_ref, k_ref, v_ref, cnt_ref, g_ref, b_ref,
              w1_ref, b1_ref, w2_ref, b2_ref, o_ref):
    i = pl.program_id(0)
    b = i // (N // BLK34)
    koff = b * (NH * NB)
    kb = k_ref[pl.ds(koff, NH * NB), :]
    vb = v_ref[pl.ds(koff, NH * NB), :]
    cntb = cnt_ref[pl.ds(koff, NH * NB), 0:1]
    bias = jnp.where(cntb >= 1.0, 0.0, NEG)
    scale = DH ** -0.5
    kbb = kb.astype(jnp.bfloat16)
    vbb = vb.astype(jnp.bfloat16)
    qbb = lax.dot_general(xn_ref[...].astype(jnp.bfloat16), wq_ref[...],
                          (((1,), (1,)), ((), ())),
                          preferred_element_type=jnp.float32
                          ).astype(jnp.bfloat16)
    outs = []
    for h in range(H):
        kh = kbb[:, h * DH:(h + 1) * DH]
        qh = qbb[:, h * DH:(h + 1) * DH]
        sT = lax.dot_general(kh, qh, (((1,), (1,)), ((), ())),
                             preferred_element_type=jnp.float32)
        sT = sT * scale + bias
        m = jnp.max(sT, axis=0, keepdims=True)
        e = jnp.exp(sT - m)
        p = (e * (1.0 / jnp.sum(e, axis=0, keepdims=True))
             ).astype(jnp.bfloat16)
        vh = vbb[:, h * DH:(h + 1) * DH]
        outs.append(lax.dot_general(p, vh, (((0,), (0,)), ((), ())),
                                    preferred_element_type=jnp.float32))
    out = jnp.concatenate(outs, axis=1)
    x2 = x_ref[...] + out
    hb = _ln(x2, g_ref[...], b_ref[...]).astype(jnp.bfloat16)
    t = lax.dot_general(hb, w1_ref[...], (((1,), (1,)), ((), ())),
                        preferred_element_type=jnp.float32)
    t = t + b1_ref[...]
    g = (0.5 * t * (1.0 + lax.erf(t * (2.0 ** -0.5)))).astype(jnp.bfloat16)
    o = lax.dot_general(g, w2_ref[...], (((1,), (1,)), ((), ())),
                        preferred_element_type=jnp.float32)
    o_ref[...] = x2 + o + b2_ref[...]


def _phase34(x2d, xn2d, wq_b, k, v, cnt, g2, b2, fc1_w, fc1_b, fc2_w,
             fc2_b):
    F = 4 * C
    return pl.pallas_call(
        _p34_body,
        grid=(R // BLK34,),
        in_specs=[
            pl.BlockSpec((BLK34, C), lambda i: (i, 0)),
            pl.BlockSpec((BLK34, C), lambda i: (i, 0)),
            pl.BlockSpec((C, C), lambda i: (0, 0)),
            pl.BlockSpec((G, C), lambda i: (0, 0)),
            pl.BlockSpec((G, C), lambda i: (0, 0)),
            pl.BlockSpec((G, 16), lambda i: (0, 0)),
            pl.BlockSpec((1, C), lambda i: (0, 0)),
            pl.BlockSpec((1, C), lambda i: (0, 0)),
            pl.BlockSpec((F, C), lambda i: (0, 0)),
            pl.BlockSpec((1, F), lambda i: (0, 0)),
            pl.BlockSpec((C, F), lambda i: (0, 0)),
            pl.BlockSpec((1, C), lambda i: (0, 0)),
        ],
        out_specs=pl.BlockSpec((BLK34, C), lambda i: (i, 0)),
        out_shape=jax.ShapeDtypeStruct((R, C), jnp.float32),
        compiler_params=pltpu.CompilerParams(
            vmem_limit_bytes=128 * 1024 * 1024),
    )(x2d, xn2d, wq_b, k, v, cnt, g2, b2, fc1_w, fc1_b, fc2_w, fc2_b)


@jax.jit
def kernel(x, rotations, norm1_g, norm1_b, Wq, Wkv, norm2_g, norm2_b,
           fc1_w, fc1_b, fc2_w, fc2_b):
    x2d = x.reshape(R, C)
    rotflat = rotations[0].reshape(C, NH * (NB // 2))
    g1 = norm1_g.reshape(1, C)
    b1 = norm1_b.reshape(1, C)
    g2 = norm2_g.reshape(1, C)
    b2 = norm2_b.reshape(1, C)

    wq_b = Wq.astype(jnp.bfloat16)
    xn2d, gid8, p23, ca = _phase1a(x2d, g1, b1, rotflat)

    pp = _sc_pool(xn2d, gid8)

    k, v, cnt, fc1_b16, fc2_b16 = _phase2(pp, p23, ca, Wkv, fc1_w, fc2_w)
    out = _phase34(x2d, xn2d, wq_b, k, v, cnt, g2, b2, fc1_b16,
                   fc1_b.reshape(1, 4 * C), fc2_b16, fc2_b.reshape(1, C))
    return out.reshape(B, N, C)


# speedup vs baseline: 1.0435x; 1.0435x over previous
import functools

import jax
import jax.numpy as jnp
from jax import lax
from jax.experimental import pallas as pl
from jax.experimental.pallas import tpu as pltpu
from jax.experimental.pallas import tpu_sc as plsc

B, N, C = 2, 4096, 1024
H, DH = 16, 64
NH, NB = 4, 8
G = B * NH * NB
R = B * N

BLK = 512
BLK34 = 512
NEG = -3.4028235e38

NC, NS = 2, 16
NW = NC * NS
RPW = R // NW
KCH = 32
NHSC = 2
NCH = RPW // KCH


def _ln(x, g, b, eps=1e-5):
    mu = jnp.mean(x, axis=-1, keepdims=True)
    var = jnp.mean((x - mu) ** 2, axis=-1, keepdims=True)
    return (x - mu) * jax.lax.rsqrt(var + eps) * g + b


def _p1a_body(x_ref, g_ref, b_ref, rot_ref, xn_ref, gid_ref, p23_ref,
              ca_ref):
    i = pl.program_id(0)
    b = i // (N // BLK)
    xn = _ln(x_ref[...], g_ref[...], b_ref[...])
    xn_ref[...] = xn
    rvT = lax.dot_general(rot_ref[...], xn, (((0,), (1,)), ((), ())),
                          preferred_element_type=jnp.float32)
    iota8 = lax.broadcasted_iota(jnp.int32, (NB, BLK), 0)
    ohs = []
    for h in range(NH):
        sub = rvT[h * (NB // 2):(h + 1) * (NB // 2), :]
        full = jnp.concatenate([sub, -sub], axis=0)
        mx = jnp.max(full, axis=0, keepdims=True)
        bid = jnp.min(jnp.where(full >= mx, iota8, NB), axis=0)
        gid_ref[h, :] = b * (NH * NB) + h * NB + bid
        ohs.append((iota8 == bid[None, :]).astype(jnp.float32))
    for h in range(NH, 8):
        gid_ref[h, :] = jnp.zeros((BLK,), jnp.int32)
    ohT = jnp.concatenate(ohs, axis=0)
    cc = jnp.broadcast_to(jnp.sum(ohT, axis=1, keepdims=True),
                          (NH * NB, 128))
    pc = lax.dot_general(ohT[NHSC * NB:].astype(jnp.bfloat16),
                         xn.astype(jnp.bfloat16), (((1,), (0,)), ((), ())),
                         preferred_element_type=jnp.float32)

    @pl.when(i % (N // BLK) == 0)
    def _init():
        p23_ref[0] = pc
        ca_ref[0] = cc

    @pl.when(i % (N // BLK) != 0)
    def _acc():
        p23_ref[0] += pc
        ca_ref[0] += cc


def _phase1a(x2d, g1, b1, rotflat):
    return pl.pallas_call(
        _p1a_body,
        grid=(R // BLK,),
        in_specs=[
            pl.BlockSpec((BLK, C), lambda i: (i, 0)),
            pl.BlockSpec((1, C), lambda i: (0, 0)),
            pl.BlockSpec((1, C), lambda i: (0, 0)),
            pl.BlockSpec((C, 16), lambda i: (0, 0)),
        ],
        out_specs=[
            pl.BlockSpec((BLK, C), lambda i: (i, 0)),
            pl.BlockSpec((8, BLK), lambda i: (0, i)),
            pl.BlockSpec((1, (NH - NHSC) * NB, C),
                         lambda i: (i // (N // BLK), 0, 0)),
            pl.BlockSpec((1, NH * NB, 128), lambda i: (i // (N // BLK), 0, 0)),
        ],
        out_shape=[
            jax.ShapeDtypeStruct((R, C), jnp.float32),
            jax.ShapeDtypeStruct((8, R), jnp.int32),
            jax.ShapeDtypeStruct((B, (NH - NHSC) * NB, C), jnp.float32),
            jax.ShapeDtypeStruct((B, NH * NB, 128), jnp.float32),
        ],
    )(x2d, g1, b1, rotflat)


def _sc_pool_body(xn_hbm, gid_hbm, outp_hbm,
                  xbuf, gb0, gb1, acc_p, sems):
    c = lax.axis_index("c")
    s = lax.axis_index("s")
    w = s * NC + c
    gbufs = (gb0, gb1)
    boff = (w // (NW // B)) * (NH * NB)

    cp0 = pltpu.async_copy(gid_hbm.at[0, pl.ds(w * RPW, RPW)], gb0,
                           sems.at[2])
    cp1 = pltpu.async_copy(gid_hbm.at[1, pl.ds(w * RPW, RPW)], gb1,
                           sems.at[3])
    first = pltpu.async_copy(xn_hbm.at[pl.ds(w * RPW, KCH)],
                             xbuf.at[0], sems.at[0])

    z16 = jnp.zeros((16,), jnp.float32)

    def zrow(r, _):
        def zcol(t, _):
            acc_p[r, pl.ds(t * 16, 16)] = z16
            return ()
        lax.fori_loop(0, C // 16, zcol, (), unroll=8)
        return ()

    lax.fori_loop(0, NHSC * NB, zrow, ())

    iota16 = lax.iota(jnp.int32, 16)
    cp0.wait()
    cp1.wait()

    for i in range(NCH):
        if i + 1 < NCH:
            nxt = pltpu.async_copy(
                xn_hbm.at[pl.ds(w * RPW + (i + 1) * KCH, KCH)],
                xbuf.at[(i + 1) % 2], sems.at[(i + 1) % 2])
        if i == 0:
            first.wait()
        else:
            prev.wait()
        for grp in range(KCH // 16):
            gvs = [gbufs[h][pl.ds(i * KCH + grp * 16, 16)].astype(jnp.float32)
                   for h in range(NHSC)]

            def row_body(l, _):
                js = [(jnp.sum(jnp.where(iota16 == l, gvs[h], 0.0))
                       .astype(jnp.int32) - boff) for h in range(NHSC)]
                r = grp * 16 + l

                def col_body(t, _):
                    x16 = xbuf[i % 2, r, pl.ds(t * 16, 16)]
                    for h in range(NHSC):
                        plsc.addupdate(acc_p.at[js[h], pl.ds(t * 16, 16)],
                                       x16)
                    return ()

                lax.fori_loop(0, C // 16, col_body, (), unroll=4)
                return ()

            lax.fori_loop(0, 16, row_body, ())
        if i + 1 < NCH:
            prev = nxt

    pltpu.sync_copy(acc_p, outp_hbm.at[w])


def _sc_pool(xn2d, gid8):
    mesh = plsc.VectorSubcoreMesh(core_axis_name="c", subcore_axis_name="s",
                                  num_cores=NC, num_subcores=NS)
    f = pl.kernel(
        _sc_pool_body,
        out_type=jax.ShapeDtypeStruct((NW, NHSC * NB, C), jnp.float32),
        mesh=mesh,
        compiler_params=pltpu.CompilerParams(needs_layout_passes=False),
        scratch_types=[
            pltpu.VMEM((2, KCH, C), jnp.float32),
            pltpu.VMEM((RPW,), jnp.int32),
            pltpu.VMEM((RPW,), jnp.int32),
            pltpu.VMEM((NHSC * NB, C), jnp.float32),
            pltpu.SemaphoreType.DMA((4,)),
        ],
    )
    return f(xn2d, gid8)


def _p2_body(pp_ref, p23_ref, ca_ref, wkv_ref, k_ref, v_ref, cnt_ref):
    halves = []
    for b in range(B):
        lo = b * (NW // B)
        acc = pp_ref[lo]
        for i in range(lo + 1, lo + NW // B):
            acc = acc + pp_ref[i]
        halves.append(jnp.concatenate([acc, p23_ref[b]], axis=0))
    pooled = jnp.concatenate(halves, axis=0)
    cnt = jnp.concatenate([ca_ref[0, :, 0:16], ca_ref[1, :, 0:16]], axis=0)
    cnt_ref[...] = cnt
    rp = pooled * (1.0 / (cnt[:, 0:1] + 1e-20))
    kv = lax.dot_general(rp, wkv_ref[...], (((1,), (1,)), ((), ())),
                         preferred_element_type=jnp.float32)
    k_ref[...] = kv[:, :C]
    v_ref[...] = kv[:, C:]


def _phase2(pp, p23, ca, Wkv):
    return pl.pallas_call(
        _p2_body,
        out_shape=[
            jax.ShapeDtypeStruct((G, C), jnp.float32),
            jax.ShapeDtypeStruct((G, C), jnp.float32),
            jax.ShapeDtypeStruct((G, 16), jnp.float32),
        ],
    )(pp, p23, ca, Wkv)


def _p34_body(x_ref, xn_ref, wq_ref, k_ref, v_ref, cnt_ref, g_ref, b_ref,
              w1_ref, b1_ref, w2_ref, b2_ref, o_ref):
    i = pl.program_id(0)
    b = i // (N // BLK34)
    koff = b * (NH * NB)
    kb = k_ref[pl.ds(koff, NH * NB), :]
    vb = v_ref[pl.ds(koff, NH * NB), :]
    cntb = cnt_ref[pl.ds(koff, NH * NB), 0:1]
    bias = jnp.where(cntb >= 1.0, 0.0, NEG)
    scale = DH ** -0.5
    kbb = kb.astype(jnp.bfloat16)
    vbb = vb.astype(jnp.bfloat16)
    qbb = lax.dot_general(xn_ref[...].astype(jnp.bfloat16), wq_ref[...],
                          (((1,), (1,)), ((), ())),
                          preferred_element_type=jnp.float32
                          ).astype(jnp.bfloat16)
    outs = []
    for h in range(H):
        kh = kbb[:, h * DH:(h + 1) * DH]
        qh = qbb[:, h * DH:(h + 1) * DH]
        sT = lax.dot_general(kh, qh, (((1,), (1,)), ((), ())),
                             preferred_element_type=jnp.float32)
        sT = sT * scale + bias
        m = jnp.max(sT, axis=0, keepdims=True)
        e = jnp.exp(sT - m)
        p = (e * (1.0 / jnp.sum(e, axis=0, keepdims=True))
             ).astype(jnp.bfloat16)
        vh = vbb[:, h * DH:(h + 1) * DH]
        outs.append(lax.dot_general(p, vh, (((0,), (0,)), ((), ())),
                                    preferred_element_type=jnp.float32))
    out = jnp.concatenate(outs, axis=1)
    x2 = x_ref[...] + out
    hb = _ln(x2, g_ref[...], b_ref[...]).astype(jnp.bfloat16)
    t = lax.dot_general(hb, w1_ref[...], (((1,), (1,)), ((), ())),
                        preferred_element_type=jnp.float32)
    t = t + b1_ref[...]
    g = (0.5 * t * (1.0 + lax.erf(t * (2.0 ** -0.5)))).astype(jnp.bfloat16)
    o = lax.dot_general(g, w2_ref[...], (((1,), (1,)), ((), ())),
                        preferred_element_type=jnp.float32)
    o_ref[...] = x2 + o + b2_ref[...]


def _phase34(x2d, xn2d, wq_b, k, v, cnt, g2, b2, fc1_w, fc1_b, fc2_w,
             fc2_b):
    F = 4 * C
    return pl.pallas_call(
        _p34_body,
        grid=(R // BLK34,),
        in_specs=[
            pl.BlockSpec((BLK34, C), lambda i: (i, 0)),
            pl.BlockSpec((BLK34, C), lambda i: (i, 0)),
            pl.BlockSpec((C, C), lambda i: (0, 0)),
            pl.BlockSpec((G, C), lambda i: (0, 0)),
            pl.BlockSpec((G, C), lambda i: (0, 0)),
            pl.BlockSpec((G, 16), lambda i: (0, 0)),
            pl.BlockSpec((1, C), lambda i: (0, 0)),
            pl.BlockSpec((1, C), lambda i: (0, 0)),
            pl.BlockSpec((F, C), lambda i: (0, 0)),
            pl.BlockSpec((1, F), lambda i: (0, 0)),
            pl.BlockSpec((C, F), lambda i: (0, 0)),
            pl.BlockSpec((1, C), lambda i: (0, 0)),
        ],
        out_specs=pl.BlockSpec((BLK34, C), lambda i: (i, 0)),
        out_shape=jax.ShapeDtypeStruct((R, C), jnp.float32),
        compiler_params=pltpu.CompilerParams(
            vmem_limit_bytes=128 * 1024 * 1024),
    )(x2d, xn2d, wq_b, k, v, cnt, g2, b2, fc1_w, fc1_b, fc2_w, fc2_b)


@jax.jit
def kernel(x, rotations, norm1_g, norm1_b, Wq, Wkv, norm2_g, norm2_b,
           fc1_w, fc1_b, fc2_w, fc2_b):
    x2d = x.reshape(R, C)
    rotflat = rotations[0].reshape(C, NH * (NB // 2))
    g1 = norm1_g.reshape(1, C)
    b1 = norm1_b.reshape(1, C)
    g2 = norm2_g.reshape(1, C)
    b2 = norm2_b.reshape(1, C)

    wq_b = Wq.astype(jnp.bfloat16)
    fc1_b16 = fc1_w.astype(jnp.bfloat16)
    fc2_b16 = fc2_w.astype(jnp.bfloat16)
    xn2d, gid8, p23, ca = _phase1a(x2d, g1, b1, rotflat)

    pp = _sc_pool(xn2d, gid8)

    k, v, cnt = _phase2(pp, p23, ca, Wkv)
    out = _phase34(x2d, xn2d, wq_b, k, v, cnt, g2, b2, fc1_b16,
                   fc1_b.reshape(1, 4 * C), fc2_b16, fc2_b.reshape(1, C))
    return out.reshape(B, N, C)


# speedup vs baseline: 1.0735x; 1.0288x over previous
import functools

import jax
import jax.numpy as jnp
from jax import lax
from jax.experimental import pallas as pl
from jax.experimental.pallas import tpu as pltpu
from jax.experimental.pallas import tpu_sc as plsc

B, N, C = 2, 4096, 1024
H, DH = 16, 64
NH, NB = 4, 8
G = B * NH * NB
R = B * N

BLK = 512
BLK34 = 512
NEG = -3.4028235e38

NC, NS = 2, 16
NW = NC * NS
RPW = R // NW
KCH = 32
NHSC = 2
NCH = RPW // KCH


def _ln(x, g, b, eps=1e-5):
    mu = jnp.mean(x, axis=-1, keepdims=True)
    var = jnp.mean((x - mu) ** 2, axis=-1, keepdims=True)
    return (x - mu) * jax.lax.rsqrt(var + eps) * g + b


def _p1a_body(x_ref, g_ref, b_ref, rot_ref, xn_ref, gid_ref, p23_ref,
              ca_ref):
    i = pl.program_id(0)
    b = i // (N // BLK)
    xn = _ln(x_ref[...], g_ref[...], b_ref[...])
    xn_ref[...] = xn
    rvT = lax.dot_general(rot_ref[...], xn, (((0,), (1,)), ((), ())),
                          preferred_element_type=jnp.float32)
    iota8 = lax.broadcasted_iota(jnp.int32, (NB, BLK), 0)
    ohs = []
    for h in range(NH):
        sub = rvT[h * (NB // 2):(h + 1) * (NB // 2), :]
        full = jnp.concatenate([sub, -sub], axis=0)
        mx = jnp.max(full, axis=0, keepdims=True)
        bid = jnp.min(jnp.where(full >= mx, iota8, NB), axis=0)
        gid_ref[h, :] = b * (NH * NB) + h * NB + bid
        ohs.append((iota8 == bid[None, :]).astype(jnp.float32))
    for h in range(NH, 8):
        gid_ref[h, :] = jnp.zeros((BLK,), jnp.int32)
    ohT = jnp.concatenate(ohs, axis=0)
    cc = jnp.broadcast_to(jnp.sum(ohT, axis=1, keepdims=True),
                          (NH * NB, 128))
    pc = lax.dot_general(ohT[NHSC * NB:].astype(jnp.bfloat16),
                         xn.astype(jnp.bfloat16), (((1,), (0,)), ((), ())),
                         preferred_element_type=jnp.float32)

    @pl.when(i % (N // BLK) == 0)
    def _init():
        p23_ref[0] = pc
        ca_ref[0] = cc

    @pl.when(i % (N // BLK) != 0)
    def _acc():
        p23_ref[0] += pc
        ca_ref[0] += cc


def _phase1a(x2d, g1, b1, rotflat):
    return pl.pallas_call(
        _p1a_body,
        grid=(R // BLK,),
        in_specs=[
            pl.BlockSpec((BLK, C), lambda i: (i, 0)),
            pl.BlockSpec((1, C), lambda i: (0, 0)),
            pl.BlockSpec((1, C), lambda i: (0, 0)),
            pl.BlockSpec((C, 16), lambda i: (0, 0)),
        ],
        out_specs=[
            pl.BlockSpec((BLK, C), lambda i: (i, 0)),
            pl.BlockSpec((8, BLK), lambda i: (0, i)),
            pl.BlockSpec((1, (NH - NHSC) * NB, C),
                         lambda i: (i // (N // BLK), 0, 0)),
            pl.BlockSpec((1, NH * NB, 128), lambda i: (i // (N // BLK), 0, 0)),
        ],
        out_shape=[
            jax.ShapeDtypeStruct((R, C), jnp.float32),
            jax.ShapeDtypeStruct((8, R), jnp.int32),
            jax.ShapeDtypeStruct((B, (NH - NHSC) * NB, C), jnp.float32),
            jax.ShapeDtypeStruct((B, NH * NB, 128), jnp.float32),
        ],
    )(x2d, g1, b1, rotflat)


def _sc_pool_body(xn_hbm, gid_hbm, outp_hbm,
                  xbuf, gb0, gb1, acc_p, sems):
    c = lax.axis_index("c")
    s = lax.axis_index("s")
    w = s * NC + c
    gbufs = (gb0, gb1)
    boff = (w // (NW // B)) * (NH * NB)

    cp0 = pltpu.async_copy(gid_hbm.at[0, pl.ds(w * RPW, RPW)], gb0,
                           sems.at[2])
    cp1 = pltpu.async_copy(gid_hbm.at[1, pl.ds(w * RPW, RPW)], gb1,
                           sems.at[3])
    first = pltpu.async_copy(xn_hbm.at[pl.ds(w * RPW, KCH)],
                             xbuf.at[0], sems.at[0])

    z16 = jnp.zeros((16,), jnp.float32)

    def zrow(r, _):
        def zcol(t, _):
            acc_p[r, pl.ds(t * 16, 16)] = z16
            return ()
        lax.fori_loop(0, C // 16, zcol, (), unroll=8)
        return ()

    lax.fori_loop(0, NHSC * NB, zrow, ())

    iota16 = lax.iota(jnp.int32, 16)
    cp0.wait()
    cp1.wait()

    for i in range(NCH):
        if i + 1 < NCH:
            nxt = pltpu.async_copy(
                xn_hbm.at[pl.ds(w * RPW + (i + 1) * KCH, KCH)],
                xbuf.at[(i + 1) % 2], sems.at[(i + 1) % 2])
        if i == 0:
            first.wait()
        else:
            prev.wait()
        for grp in range(KCH // 16):
            gvs = [gbufs[h][pl.ds(i * KCH + grp * 16, 16)].astype(jnp.float32)
                   for h in range(NHSC)]

            def row_body(l, _):
                js = [(jnp.sum(jnp.where(iota16 == l, gvs[h], 0.0))
                       .astype(jnp.int32) - boff) for h in range(NHSC)]
                r = grp * 16 + l

                def col_body(t, _):
                    x16 = xbuf[i % 2, r, pl.ds(t * 16, 16)]
                    for h in range(NHSC):
                        plsc.addupdate(acc_p.at[js[h], pl.ds(t * 16, 16)],
                                       x16)
                    return ()

                lax.fori_loop(0, C // 16, col_body, (), unroll=4)
                return ()

            lax.fori_loop(0, 16, row_body, ())
        if i + 1 < NCH:
            prev = nxt

    pltpu.sync_copy(acc_p, outp_hbm.at[w])


def _sc_pool(xn2d, gid8):
    mesh = plsc.VectorSubcoreMesh(core_axis_name="c", subcore_axis_name="s",
                                  num_cores=NC, num_subcores=NS)
    f = pl.kernel(
        _sc_pool_body,
        out_type=jax.ShapeDtypeStruct((NW, NHSC * NB, C), jnp.float32),
        mesh=mesh,
        compiler_params=pltpu.CompilerParams(needs_layout_passes=False),
        scratch_types=[
            pltpu.VMEM((2, KCH, C), jnp.float32),
            pltpu.VMEM((RPW,), jnp.int32),
            pltpu.VMEM((RPW,), jnp.int32),
            pltpu.VMEM((NHSC * NB, C), jnp.float32),
            pltpu.SemaphoreType.DMA((4,)),
        ],
    )
    return f(xn2d, gid8)


def _p1b_body(xn_ref, wq_ref, q_ref):
    q_ref[...] = lax.dot_general(xn_ref[...].astype(jnp.bfloat16), wq_ref[...],
                                 (((1,), (1,)), ((), ())),
                                 preferred_element_type=jnp.float32
                                 ).astype(jnp.bfloat16)


def _phase1b(xn2d, wq_b):
    return pl.pallas_call(
        _p1b_body,
        grid=(R // BLK,),
        in_specs=[
            pl.BlockSpec((BLK, C), lambda i: (i, 0)),
            pl.BlockSpec((C, C), lambda i: (0, 0)),
        ],
        out_specs=pl.BlockSpec((BLK, C), lambda i: (i, 0)),
        out_shape=jax.ShapeDtypeStruct((R, C), jnp.bfloat16),
    )(xn2d, wq_b)


def _p2_body(pp_ref, p23_ref, ca_ref, wkv_ref, k_ref, v_ref, cnt_ref):
    halves = []
    for b in range(B):
        lo = b * (NW // B)
        acc = pp_ref[lo]
        for i in range(lo + 1, lo + NW // B):
            acc = acc + pp_ref[i]
        halves.append(jnp.concatenate([acc, p23_ref[b]], axis=0))
    pooled = jnp.concatenate(halves, axis=0)
    cnt = jnp.concatenate([ca_ref[0, :, 0:16], ca_ref[1, :, 0:16]], axis=0)
    cnt_ref[...] = cnt
    rp = pooled * (1.0 / (cnt[:, 0:1] + 1e-20))
    kv = lax.dot_general(rp, wkv_ref[...], (((1,), (1,)), ((), ())),
                         preferred_element_type=jnp.float32)
    k_ref[...] = kv[:, :C]
    v_ref[...] = kv[:, C:]


def _phase2(pp, p23, ca, Wkv):
    return pl.pallas_call(
        _p2_body,
        out_shape=[
            jax.ShapeDtypeStruct((G, C), jnp.float32),
            jax.ShapeDtypeStruct((G, C), jnp.float32),
            jax.ShapeDtypeStruct((G, 16), jnp.float32),
        ],
    )(pp, p23, ca, Wkv)


def _p34_body(x_ref, q_ref, k_ref, v_ref, cnt_ref, g_ref, b_ref,
              w1_ref, b1_ref, w2_ref, b2_ref, o_ref):
    i = pl.program_id(0)
    b = i // (N // BLK34)
    koff = b * (NH * NB)
    kb = k_ref[pl.ds(koff, NH * NB), :]
    vb = v_ref[pl.ds(koff, NH * NB), :]
    cntb = cnt_ref[pl.ds(koff, NH * NB), 0:1]
    bias = jnp.where(cntb >= 1.0, 0.0, NEG)
    scale = DH ** -0.5
    kbb = kb.astype(jnp.bfloat16)
    vbb = vb.astype(jnp.bfloat16)
    qbb = q_ref[...]
    outs = []
    for h in range(H):
        kh = kbb[:, h * DH:(h + 1) * DH]
        qh = qbb[:, h * DH:(h + 1) * DH]
        sT = lax.dot_general(kh, qh, (((1,), (1,)), ((), ())),
                             preferred_element_type=jnp.float32)
        sT = sT * scale + bias
        m = jnp.max(sT, axis=0, keepdims=True)
        e = jnp.exp(sT - m)
        p = (e * (1.0 / jnp.sum(e, axis=0, keepdims=True))
             ).astype(jnp.bfloat16)
        vh = vbb[:, h * DH:(h + 1) * DH]
        outs.append(lax.dot_general(p, vh, (((0,), (0,)), ((), ())),
                                    preferred_element_type=jnp.float32))
    out = jnp.concatenate(outs, axis=1)
    x2 = x_ref[...] + out
    hb = _ln(x2, g_ref[...], b_ref[...]).astype(jnp.bfloat16)
    t = lax.dot_general(hb, w1_ref[...], (((1,), (1,)), ((), ())),
                        preferred_element_type=jnp.float32)
    t = t + b1_ref[...]
    g = (0.5 * t * (1.0 + lax.erf(t * (2.0 ** -0.5)))).astype(jnp.bfloat16)
    o = lax.dot_general(g, w2_ref[...], (((1,), (1,)), ((), ())),
                        preferred_element_type=jnp.float32)
    o_ref[...] = x2 + o + b2_ref[...]


def _phase34(x2d, q2d, k, v, cnt, g2, b2, fc1_w, fc1_b, fc2_w,
             fc2_b):
    F = 4 * C
    return pl.pallas_call(
        _p34_body,
        grid=(R // BLK34,),
        in_specs=[
            pl.BlockSpec((BLK34, C), lambda i: (i, 0)),
            pl.BlockSpec((BLK34, C), lambda i: (i, 0)),
            pl.BlockSpec((G, C), lambda i: (0, 0)),
            pl.BlockSpec((G, C), lambda i: (0, 0)),
            pl.BlockSpec((G, 16), lambda i: (0, 0)),
            pl.BlockSpec((1, C), lambda i: (0, 0)),
            pl.BlockSpec((1, C), lambda i: (0, 0)),
            pl.BlockSpec((F, C), lambda i: (0, 0)),
            pl.BlockSpec((1, F), lambda i: (0, 0)),
            pl.BlockSpec((C, F), lambda i: (0, 0)),
            pl.BlockSpec((1, C), lambda i: (0, 0)),
        ],
        out_specs=pl.BlockSpec((BLK34, C), lambda i: (i, 0)),
        out_shape=jax.ShapeDtypeStruct((R, C), jnp.float32),
        compiler_params=pltpu.CompilerParams(
            vmem_limit_bytes=128 * 1024 * 1024),
    )(x2d, q2d, k, v, cnt, g2, b2, fc1_w, fc1_b, fc2_w, fc2_b)


@jax.jit
def kernel(x, rotations, norm1_g, norm1_b, Wq, Wkv, norm2_g, norm2_b,
           fc1_w, fc1_b, fc2_w, fc2_b):
    x2d = x.reshape(R, C)
    rotflat = rotations[0].reshape(C, NH * (NB // 2))
    g1 = norm1_g.reshape(1, C)
    b1 = norm1_b.reshape(1, C)
    g2 = norm2_g.reshape(1, C)
    b2 = norm2_b.reshape(1, C)

    wq_b = Wq.astype(jnp.bfloat16)
    fc1_b16 = fc1_w.astype(jnp.bfloat16)
    fc2_b16 = fc2_w.astype(jnp.bfloat16)
    xn2d, gid8, p23, ca = _phase1a(x2d, g1, b1, rotflat)

    pp = _sc_pool(xn2d, gid8)

    q2d = _phase1b(xn2d, wq_b)
    k, v, cnt = _phase2(pp, p23, ca, Wkv)
    out = _phase34(x2d, q2d, k, v, cnt, g2, b2, fc1_b16,
                   fc1_b.reshape(1, 4 * C), fc2_b16, fc2_b.reshape(1, C))
    return out.reshape(B, N, C)


# speedup vs baseline: 1.1411x; 1.0630x over previous
import functools

import jax
import jax.numpy as jnp
from jax import lax
from jax.experimental import pallas as pl
from jax.experimental.pallas import tpu as pltpu
from jax.experimental.pallas import tpu_sc as plsc

B, N, C = 2, 4096, 1024
H, DH = 16, 64
NH, NB = 4, 8
G = B * NH * NB
R = B * N

BLK = 512
BLK34 = 512
NEG = -3.4028235e38

NC, NS = 2, 16
NW = NC * NS
RPW = R // NW
KCH = 32
NHSC = 2
NCH = RPW // KCH


def _ln(x, g, b, eps=1e-5):
    mu = jnp.mean(x, axis=-1, keepdims=True)
    var = jnp.mean((x - mu) ** 2, axis=-1, keepdims=True)
    return (x - mu) * jax.lax.rsqrt(var + eps) * g + b


def _p1a_body(x_ref, g_ref, b_ref, rot_ref, xn_ref, gid_ref, p23_ref,
              ca_ref):
    i = pl.program_id(0)
    b = i // (N // BLK)
    xn = _ln(x_ref[...], g_ref[...], b_ref[...])
    xn_ref[...] = xn
    rvT = lax.dot_general(rot_ref[...], xn, (((0,), (1,)), ((), ())),
                          preferred_element_type=jnp.float32)
    iota8 = lax.broadcasted_iota(jnp.int32, (NB, BLK), 0)
    ohs = []
    for h in range(NH):
        sub = rvT[h * (NB // 2):(h + 1) * (NB // 2), :]
        full = jnp.concatenate([sub, -sub], axis=0)
        mx = jnp.max(full, axis=0, keepdims=True)
        bid = jnp.min(jnp.where(full >= mx, iota8, NB), axis=0)
        gid_ref[h, :] = b * (NH * NB) + h * NB + bid
        ohs.append((iota8 == bid[None, :]).astype(jnp.float32))
    for h in range(NH, 8):
        gid_ref[h, :] = jnp.zeros((BLK,), jnp.int32)
    ohT = jnp.concatenate(ohs, axis=0)
    cc = jnp.broadcast_to(jnp.sum(ohT, axis=1, keepdims=True),
                          (NH * NB, 128))
    pc = lax.dot_general(ohT[NHSC * NB:].astype(jnp.bfloat16),
                         xn.astype(jnp.bfloat16), (((1,), (0,)), ((), ())),
                         preferred_element_type=jnp.float32)

    @pl.when(i % (N // BLK) == 0)
    def _init():
        p23_ref[0] = pc
        ca_ref[0] = cc

    @pl.when(i % (N // BLK) != 0)
    def _acc():
        p23_ref[0] += pc
        ca_ref[0] += cc


def _phase1a(x2d, g1, b1, rotflat):
    return pl.pallas_call(
        _p1a_body,
        grid=(R // BLK,),
        in_specs=[
            pl.BlockSpec((BLK, C), lambda i: (i, 0)),
            pl.BlockSpec((1, C), lambda i: (0, 0)),
            pl.BlockSpec((1, C), lambda i: (0, 0)),
            pl.BlockSpec((C, 16), lambda i: (0, 0)),
        ],
        out_specs=[
            pl.BlockSpec((BLK, C), lambda i: (i, 0)),
            pl.BlockSpec((8, BLK), lambda i: (0, i)),
            pl.BlockSpec((1, (NH - NHSC) * NB, C),
                         lambda i: (i // (N // BLK), 0, 0)),
            pl.BlockSpec((1, NH * NB, 128), lambda i: (i // (N // BLK), 0, 0)),
        ],
        out_shape=[
            jax.ShapeDtypeStruct((R, C), jnp.float32),
            jax.ShapeDtypeStruct((8, R), jnp.int32),
            jax.ShapeDtypeStruct((B, (NH - NHSC) * NB, C), jnp.float32),
            jax.ShapeDtypeStruct((B, NH * NB, 128), jnp.float32),
        ],
    )(x2d, g1, b1, rotflat)


def _sc_pool_body(xn_hbm, gid_hbm, outp_hbm,
                  xbuf, gb0, gb1, acc_p, sems):
    c = lax.axis_index("c")
    s = lax.axis_index("s")
    w = s * NC + c
    gbufs = (gb0, gb1)
    boff = (w // (NW // B)) * (NH * NB)

    cp0 = pltpu.async_copy(gid_hbm.at[0, pl.ds(w * RPW, RPW)], gb0,
                           sems.at[2])
    cp1 = pltpu.async_copy(gid_hbm.at[1, pl.ds(w * RPW, RPW)], gb1,
                           sems.at[3])
    first = pltpu.async_copy(xn_hbm.at[pl.ds(w * RPW, KCH)],
                             xbuf.at[0], sems.at[0])

    z16 = jnp.zeros((16,), jnp.float32)

    def zrow(r, _):
        def zcol(t, _):
            acc_p[r, pl.ds(t * 16, 16)] = z16
            return ()
        lax.fori_loop(0, C // 16, zcol, (), unroll=8)
        return ()

    lax.fori_loop(0, NHSC * NB, zrow, ())

    iota16 = lax.iota(jnp.int32, 16)
    cp0.wait()
    cp1.wait()

    for i in range(NCH):
        if i + 1 < NCH:
            nxt = pltpu.async_copy(
                xn_hbm.at[pl.ds(w * RPW + (i + 1) * KCH, KCH)],
                xbuf.at[(i + 1) % 2], sems.at[(i + 1) % 2])
        if i == 0:
            first.wait()
        else:
            prev.wait()
        for grp in range(KCH // 16):
            gvs = [gbufs[h][pl.ds(i * KCH + grp * 16, 16)].astype(jnp.float32)
                   for h in range(NHSC)]

            def row_body(l, _):
                js = [(jnp.sum(jnp.where(iota16 == l, gvs[h], 0.0))
                       .astype(jnp.int32) - boff) for h in range(NHSC)]
                r = grp * 16 + l

                @functools.partial(plsc.parallel_loop, 0, C // 16,
                                   unroll=4)
                def col_body(t):
                    x16 = xbuf[i % 2, r, pl.ds(t * 16, 16)]
                    for h in range(NHSC):
                        plsc.addupdate(acc_p.at[js[h], pl.ds(t * 16, 16)],
                                       x16)

                return ()

            lax.fori_loop(0, 16, row_body, ())
        if i + 1 < NCH:
            prev = nxt

    pltpu.sync_copy(acc_p, outp_hbm.at[w])


def _sc_pool(xn2d, gid8):
    mesh = plsc.VectorSubcoreMesh(core_axis_name="c", subcore_axis_name="s",
                                  num_cores=NC, num_subcores=NS)
    f = pl.kernel(
        _sc_pool_body,
        out_type=jax.ShapeDtypeStruct((NW, NHSC * NB, C), jnp.float32),
        mesh=mesh,
        compiler_params=pltpu.CompilerParams(needs_layout_passes=False),
        scratch_types=[
            pltpu.VMEM((2, KCH, C), jnp.float32),
            pltpu.VMEM((RPW,), jnp.int32),
            pltpu.VMEM((RPW,), jnp.int32),
            pltpu.VMEM((NHSC * NB, C), jnp.float32),
            pltpu.SemaphoreType.DMA((4,)),
        ],
    )
    return f(xn2d, gid8)


def _p1b_body(xn_ref, wq_ref, q_ref):
    q_ref[...] = lax.dot_general(xn_ref[...].astype(jnp.bfloat16), wq_ref[...],
                                 (((1,), (1,)), ((), ())),
                                 preferred_element_type=jnp.float32
                                 ).astype(jnp.bfloat16)


def _phase1b(xn2d, wq_b):
    return pl.pallas_call(
        _p1b_body,
        grid=(R // BLK,),
        in_specs=[
            pl.BlockSpec((BLK, C), lambda i: (i, 0)),
            pl.BlockSpec((C, C), lambda i: (0, 0)),
        ],
        out_specs=pl.BlockSpec((BLK, C), lambda i: (i, 0)),
        out_shape=jax.ShapeDtypeStruct((R, C), jnp.bfloat16),
    )(xn2d, wq_b)


def _p2_body(pp_ref, p23_ref, ca_ref, wkv_ref, k_ref, v_ref, cnt_ref):
    halves = []
    for b in range(B):
        lo = b * (NW // B)
        acc = pp_ref[lo]
        for i in range(lo + 1, lo + NW // B):
            acc = acc + pp_ref[i]
        halves.append(jnp.concatenate([acc, p23_ref[b]], axis=0))
    pooled = jnp.concatenate(halves, axis=0)
    cnt = jnp.concatenate([ca_ref[0, :, 0:16], ca_ref[1, :, 0:16]], axis=0)
    cnt_ref[...] = cnt
    rp = pooled * (1.0 / (cnt[:, 0:1] + 1e-20))
    kv = lax.dot_general(rp, wkv_ref[...], (((1,), (1,)), ((), ())),
                         preferred_element_type=jnp.float32)
    k_ref[...] = kv[:, :C]
    v_ref[...] = kv[:, C:]


def _phase2(pp, p23, ca, Wkv):
    return pl.pallas_call(
        _p2_body,
        out_shape=[
            jax.ShapeDtypeStruct((G, C), jnp.float32),
            jax.ShapeDtypeStruct((G, C), jnp.float32),
            jax.ShapeDtypeStruct((G, 16), jnp.float32),
        ],
    )(pp, p23, ca, Wkv)


def _p34_body(x_ref, q_ref, k_ref, v_ref, cnt_ref, g_ref, b_ref,
              w1_ref, b1_ref, w2_ref, b2_ref, o_ref):
    i = pl.program_id(0)
    b = i // (N // BLK34)
    koff = b * (NH * NB)
    kb = k_ref[pl.ds(koff, NH * NB), :]
    vb = v_ref[pl.ds(koff, NH * NB), :]
    cntb = cnt_ref[pl.ds(koff, NH * NB), 0:1]
    bias = jnp.where(cntb >= 1.0, 0.0, NEG)
    scale = DH ** -0.5
    kbb = kb.astype(jnp.bfloat16)
    vbb = vb.astype(jnp.bfloat16)
    qbb = q_ref[...]
    outs = []
    for h in range(H):
        kh = kbb[:, h * DH:(h + 1) * DH]
        qh = qbb[:, h * DH:(h + 1) * DH]
        sT = lax.dot_general(kh, qh, (((1,), (1,)), ((), ())),
                             preferred_element_type=jnp.float32)
        sT = sT * scale + bias
        m = jnp.max(sT, axis=0, keepdims=True)
        e = jnp.exp(sT - m)
        p = (e * (1.0 / jnp.sum(e, axis=0, keepdims=True))
             ).astype(jnp.bfloat16)
        vh = vbb[:, h * DH:(h + 1) * DH]
        outs.append(lax.dot_general(p, vh, (((0,), (0,)), ((), ())),
                                    preferred_element_type=jnp.float32))
    out = jnp.concatenate(outs, axis=1)
    x2 = x_ref[...] + out
    hb = _ln(x2, g_ref[...], b_ref[...]).astype(jnp.bfloat16)
    t = lax.dot_general(hb, w1_ref[...], (((1,), (1,)), ((), ())),
                        preferred_element_type=jnp.float32)
    t = t + b1_ref[...]
    g = (0.5 * t * (1.0 + lax.erf(t * (2.0 ** -0.5)))).astype(jnp.bfloat16)
    o = lax.dot_general(g, w2_ref[...], (((1,), (1,)), ((), ())),
                        preferred_element_type=jnp.float32)
    o_ref[...] = x2 + o + b2_ref[...]


def _phase34(x2d, q2d, k, v, cnt, g2, b2, fc1_w, fc1_b, fc2_w,
             fc2_b):
    F = 4 * C
    return pl.pallas_call(
        _p34_body,
        grid=(R // BLK34,),
        in_specs=[
            pl.BlockSpec((BLK34, C), lambda i: (i, 0)),
            pl.BlockSpec((BLK34, C), lambda i: (i, 0)),
            pl.BlockSpec((G, C), lambda i: (0, 0)),
            pl.BlockSpec((G, C), lambda i: (0, 0)),
            pl.BlockSpec((G, 16), lambda i: (0, 0)),
            pl.BlockSpec((1, C), lambda i: (0, 0)),
            pl.BlockSpec((1, C), lambda i: (0, 0)),
            pl.BlockSpec((F, C), lambda i: (0, 0)),
            pl.BlockSpec((1, F), lambda i: (0, 0)),
            pl.BlockSpec((C, F), lambda i: (0, 0)),
            pl.BlockSpec((1, C), lambda i: (0, 0)),
        ],
        out_specs=pl.BlockSpec((BLK34, C), lambda i: (i, 0)),
        out_shape=jax.ShapeDtypeStruct((R, C), jnp.float32),
        compiler_params=pltpu.CompilerParams(
            vmem_limit_bytes=128 * 1024 * 1024),
    )(x2d, q2d, k, v, cnt, g2, b2, fc1_w, fc1_b, fc2_w, fc2_b)


@jax.jit
def kernel(x, rotations, norm1_g, norm1_b, Wq, Wkv, norm2_g, norm2_b,
           fc1_w, fc1_b, fc2_w, fc2_b):
    x2d = x.reshape(R, C)
    rotflat = rotations[0].reshape(C, NH * (NB // 2))
    g1 = norm1_g.reshape(1, C)
    b1 = norm1_b.reshape(1, C)
    g2 = norm2_g.reshape(1, C)
    b2 = norm2_b.reshape(1, C)

    wq_b = Wq.astype(jnp.bfloat16)
    fc1_b16 = fc1_w.astype(jnp.bfloat16)
    fc2_b16 = fc2_w.astype(jnp.bfloat16)
    xn2d, gid8, p23, ca = _phase1a(x2d, g1, b1, rotflat)

    pp = _sc_pool(xn2d, gid8)

    q2d = _phase1b(xn2d, wq_b)
    k, v, cnt = _phase2(pp, p23, ca, Wkv)
    out = _phase34(x2d, q2d, k, v, cnt, g2, b2, fc1_b16,
                   fc1_b.reshape(1, 4 * C), fc2_b16, fc2_b.reshape(1, C))
    return out.reshape(B, N, C)
